# Initial kernel scaffold; baseline (speedup 1.0000x reference)
import jax
import jax.numpy as jnp
from jax.experimental import pallas as pl


def kernel(x, edge_index, batch, W1, b1, W2, b2, Wfc, bfc):
    raise NotImplementedError("write your pallas kernel here")


# speedup vs baseline: 13.0743x; 13.0743x over previous
import functools

import jax
import jax.numpy as jnp
from jax import lax
from jax.experimental import pallas as pl
from jax.experimental.pallas import tpu as pltpu
from jax.experimental.pallas import tpu_sc as plsc

N = 10000
E = 320000
F = 128

NC = 2
NS = 16
NW = NC * NS
EPW = E // NW
CH = 80
NIT = EPW // CH
NP = 10240
RPT = NP // NS
ZB = 128

_mesh = plsc.VectorSubcoreMesh(
    core_axis_name="c", subcore_axis_name="s", num_cores=NC, num_subcores=NS
)


@functools.partial(
    pl.kernel,
    mesh=_mesh,
    out_type=jax.ShapeDtypeStruct((NC, NP, 16), jnp.float32),
    scratch_types=[
        pltpu.VMEM_SHARED((NP, 16), jnp.float32),
        pltpu.VMEM((CH,), jnp.int32),
        pltpu.VMEM((CH, 16), jnp.float32),
        pltpu.VMEM((ZB, 16), jnp.float32),
    ],
)
def _deg_kernel(dst_hbm, out_hbm, acc_sh, dsti_v, ones_v, zero_v):
    c = lax.axis_index("c")
    s = lax.axis_index("s")
    wid = s * NC + c

    @pl.loop(0, ZB)
    def _(r):
        zero_v[r, :] = jnp.zeros((16,), jnp.float32)

    @pl.loop(0, CH)
    def _(r):
        ones_v[r, :] = jnp.full((16,), 1.0, jnp.float32)

    for k in range(RPT // ZB):
        pltpu.sync_copy(zero_v, acc_sh.at[pl.ds(s * RPT + k * ZB, ZB)])
    plsc.subcore_barrier()

    @pl.loop(0, NIT)
    def _(j):
        base = wid * EPW + j * CH
        pltpu.sync_copy(dst_hbm.at[pl.ds(base, CH)], dsti_v)
        pltpu.sync_copy(ones_v, acc_sh.at[dsti_v], add=True)

    plsc.subcore_barrier()
    for k in range(RPT // ZB):
        off = s * RPT + k * ZB
        pltpu.sync_copy(acc_sh.at[pl.ds(off, ZB)], out_hbm.at[c, pl.ds(off, ZB)])


@functools.partial(
    pl.kernel,
    mesh=_mesh,
    out_type=jax.ShapeDtypeStruct((NC, NP, F), jnp.float32),
    scratch_types=[
        pltpu.VMEM_SHARED((NP, F), jnp.float32),
        pltpu.VMEM((CH,), jnp.int32),
        pltpu.VMEM((CH,), jnp.int32),
        pltpu.VMEM((CH, F), jnp.float32),
        pltpu.VMEM((ZB, F), jnp.float32),
        pltpu.SemaphoreType.DMA,
    ],
)
def _scatter_kernel(y_hbm, src_hbm, dst_hbm, out_hbm,
                    acc_sh, srci_v, dsti_v, rows_v, zero_v, sem):
    c = lax.axis_index("c")
    s = lax.axis_index("s")
    wid = s * NC + c

    @pl.loop(0, ZB)
    def _(r):
        for cix in range(F // 16):
            zero_v[r, pl.ds(cix * 16, 16)] = jnp.zeros((16,), jnp.float32)

    for k in range(RPT // ZB):
        pltpu.sync_copy(zero_v, acc_sh.at[pl.ds(s * RPT + k * ZB, ZB)])
    plsc.subcore_barrier()

    @pl.loop(0, NIT)
    def _(j):
        base = wid * EPW + j * CH
        pltpu.sync_copy(src_hbm.at[pl.ds(base, CH)], srci_v)
        pltpu.sync_copy(dst_hbm.at[pl.ds(base, CH)], dsti_v)
        pltpu.async_copy(y_hbm.at[srci_v], rows_v, sem).wait()
        pltpu.sync_copy(rows_v, acc_sh.at[dsti_v], add=True)

    plsc.subcore_barrier()
    for k in range(RPT // ZB):
        off = s * RPT + k * ZB
        pltpu.sync_copy(acc_sh.at[pl.ds(off, ZB)], out_hbm.at[c, pl.ds(off, ZB)])


BLK = 1000
GRID = N // BLK


def _dinv_of(dp):
    deg = dp[0, :, 0] + dp[1, :, 0] + 1.0
    return lax.rsqrt(deg)


def _mm(a, b):
    return jnp.dot(a, b, preferred_element_type=jnp.float32,
                   precision=lax.Precision.HIGHEST)


def _prep_body(x_ref, w1_ref, dp_ref, xw_ref, y_ref):
    xw = _mm(x_ref[...], w1_ref[...])
    dinv = _dinv_of(dp_ref[...])
    xw_ref[...] = xw
    y_ref[...] = xw * dinv[:, None]


def _prep_call(x, W1, deg_parts):
    return pl.pallas_call(
        _prep_body,
        grid=(GRID,),
        in_specs=[
            pl.BlockSpec((BLK, F), lambda i: (i, 0)),
            pl.BlockSpec((F, F), lambda i: (0, 0)),
            pl.BlockSpec((NC, BLK, 16), lambda i: (0, i, 0)),
        ],
        out_specs=[
            pl.BlockSpec((BLK, F), lambda i: (i, 0)),
            pl.BlockSpec((BLK, F), lambda i: (i, 0)),
        ],
        out_shape=[
            jax.ShapeDtypeStruct((N, F), jnp.float32),
            jax.ShapeDtypeStruct((N, F), jnp.float32),
        ],
    )(x, W1, deg_parts)


def _mid_body(acc_ref, xw_ref, dp_ref, b_ref, w2_ref, xw2_ref, y2_ref):
    dinv = _dinv_of(dp_ref[...])[:, None]
    acc = acc_ref[0] + acc_ref[1]
    h = jax.nn.relu(dinv * acc + dinv * dinv * xw_ref[...] + b_ref[...])
    xw2 = _mm(h, w2_ref[...])
    xw2_ref[...] = xw2
    y2_ref[...] = xw2 * dinv


def _mid_call(acc1, xw1, deg_parts, b1, W2):
    return pl.pallas_call(
        _mid_body,
        grid=(GRID,),
        in_specs=[
            pl.BlockSpec((NC, BLK, F), lambda i: (0, i, 0)),
            pl.BlockSpec((BLK, F), lambda i: (i, 0)),
            pl.BlockSpec((NC, BLK, 16), lambda i: (0, i, 0)),
            pl.BlockSpec((1, F), lambda i: (0, 0)),
            pl.BlockSpec((F, F), lambda i: (0, 0)),
        ],
        out_specs=[
            pl.BlockSpec((BLK, F), lambda i: (i, 0)),
            pl.BlockSpec((BLK, F), lambda i: (i, 0)),
        ],
        out_shape=[
            jax.ShapeDtypeStruct((N, F), jnp.float32),
            jax.ShapeDtypeStruct((N, F), jnp.float32),
        ],
    )(acc1, xw1, deg_parts, b1.reshape(1, F), W2)


def _fin_body(acc_ref, xw_ref, dp_ref, b_ref, wfc_ref, bfc_ref, out_ref, sum_v):
    i = pl.program_id(0)
    dinv = _dinv_of(dp_ref[...])[:, None]
    acc = acc_ref[0] + acc_ref[1]
    h = jax.nn.relu(dinv * acc + dinv * dinv * xw_ref[...] + b_ref[...])
    part = jnp.sum(h, axis=0, keepdims=True)

    @pl.when(i == 0)
    def _():
        sum_v[...] = part

    @pl.when(i > 0)
    def _():
        sum_v[...] = sum_v[...] + part

    @pl.when(i == GRID - 1)
    def _():
        pooled = sum_v[...] * (1.0 / N)
        out_ref[...] = _mm(pooled, wfc_ref[...]) + bfc_ref[...]


def _fin_call(acc2, xw2, deg_parts, b2, Wfc, bfc):
    out_ch = Wfc.shape[1]
    return pl.pallas_call(
        _fin_body,
        grid=(GRID,),
        in_specs=[
            pl.BlockSpec((NC, BLK, F), lambda i: (0, i, 0)),
            pl.BlockSpec((BLK, F), lambda i: (i, 0)),
            pl.BlockSpec((NC, BLK, 16), lambda i: (0, i, 0)),
            pl.BlockSpec((1, F), lambda i: (0, 0)),
            pl.BlockSpec((F, out_ch), lambda i: (0, 0)),
            pl.BlockSpec((1, out_ch), lambda i: (0, 0)),
        ],
        out_specs=pl.BlockSpec((1, out_ch), lambda i: (0, 0)),
        out_shape=jax.ShapeDtypeStruct((1, out_ch), jnp.float32),
        scratch_shapes=[pltpu.VMEM((1, F), jnp.float32)],
    )(acc2, xw2, deg_parts, b2.reshape(1, F), Wfc, bfc.reshape(1, out_ch))


def kernel(x, edge_index, batch, W1, b1, W2, b2, Wfc, bfc):
    src = edge_index[0]
    dst = edge_index[1]

    deg_parts = _deg_kernel(dst)
    xw1, y1 = _prep_call(x, W1, deg_parts)
    acc1 = _scatter_kernel(y1, src, dst)
    xw2, y2 = _mid_call(acc1, xw1, deg_parts, b1, W2)
    acc2 = _scatter_kernel(y2, src, dst)
    return _fin_call(acc2, xw2, deg_parts, b2, Wfc, bfc)
